# Initial kernel scaffold; baseline (speedup 1.0000x reference)
import jax
import jax.numpy as jnp
from jax.experimental import pallas as pl


def kernel(query_points, neural_xyz, points_embedding, camrotc2w, campos, sampled_Rw2c):
    raise NotImplementedError("write your pallas kernel here")


# speedup vs baseline: 1.1526x; 1.1526x over previous
import numpy as np
import jax
import jax.numpy as jnp
from jax.experimental import pallas as pl

K_NEIGHBORS = 8
DIST_XYZ_FREQ = 5
NUM_FEAT_FREQS = 3
DIST_XYZ_DENO = 1.0
VSIZE = np.array([0.004, 0.004, 0.004], dtype=np.float32)


def _pe(positions, freqs):
    freq_bands = 2.0 ** jnp.arange(freqs, dtype=positions.dtype)
    pts = (positions[..., None] * freq_bands).reshape(positions.shape[:-1] + (freqs * positions.shape[-1],))
    return jnp.concatenate([jnp.sin(pts), jnp.cos(pts)], axis=-1)


def _w2pers(point_xyz, camrotc2w, campos):
    point_xyz_shift = point_xyz[None, ...] - campos[:, None, :]
    xyz = jnp.sum(camrotc2w[:, None, :, :] * point_xyz_shift[:, :, :, None], axis=-2)
    xper = xyz[:, :, 0] / xyz[:, :, 2]
    yper = xyz[:, :, 1] / xyz[:, :, 2]
    return jnp.stack([xper, yper, xyz[:, :, 2]], axis=-1)


def kernel(query_points, neural_xyz, points_embedding, camrotc2w, campos, sampled_Rw2c):
    Q = query_points.shape[0]
    q_sq = jnp.sum(query_points * query_points, axis=-1, keepdims=True)
    p_sq = jnp.sum(neural_xyz * neural_xyz, axis=-1)
    qb = query_points.astype(jnp.bfloat16).astype(jnp.float32)
    pb = neural_xyz.astype(jnp.bfloat16).astype(jnp.float32)
    qp = (qb[:, 0:1] * pb[None, :, 0] + qb[:, 1:2] * pb[None, :, 1]) + qb[:, 2:3] * pb[None, :, 2]
    d2 = q_sq + p_sq[None, :] - 2.0 * qp
    _, assign_index = jax.lax.top_k(-d2, K_NEIGHBORS)
    ref_xyz = jnp.take(neural_xyz, assign_index, axis=0)
    ref_fea = jnp.take(points_embedding, assign_index, axis=0)
    query_points_local = _w2pers(query_points, camrotc2w, campos)
    ref_xyz_pers = _w2pers(ref_xyz.reshape(-1, 3), camrotc2w, campos).reshape(1, Q, K_NEIGHBORS, 3)
    xdist = ref_xyz_pers[..., 0] * ref_xyz_pers[..., 2] - query_points_local[:, :, None, 0] * query_points_local[:, :, None, 2]
    ydist = ref_xyz_pers[..., 1] * ref_xyz_pers[..., 2] - query_points_local[:, :, None, 1] * query_points_local[:, :, None, 2]
    zdist = ref_xyz_pers[..., 2] - query_points_local[:, :, None, 2]
    dists = jnp.stack([xdist, ydist, zdist], axis=-1)
    dists = jnp.concatenate([ref_xyz[None, ...] - query_points[None, :, None, :], dists], axis=-1)
    dists_flat = dists.reshape(-1, dists.shape[-1])
    deno = DIST_XYZ_DENO * float(np.linalg.norm(VSIZE))
    dists_flat = dists_flat / deno
    dists_flat = dists_flat.at[:, :3].set(dists_flat[:, :3] @ sampled_Rw2c)
    dists_flat = _pe(dists_flat, DIST_XYZ_FREQ)
    ref_fea = jnp.concatenate([ref_fea, _pe(ref_fea, NUM_FEAT_FREQS)], axis=-1)
    agg_in = jnp.concatenate([ref_fea, dists_flat.reshape(Q, K_NEIGHBORS, dists_flat.shape[-1])], axis=-1)
    return agg_in


# speedup vs baseline: 3.0305x; 2.6293x over previous
import functools

import numpy as np
import jax
import jax.numpy as jnp
from jax import lax
from jax.experimental import pallas as pl
from jax.experimental.pallas import tpu as pltpu
from jax.experimental.pallas import tpu_sc as plsc

K = 8
DIST_FREQ = 5
FEAT_FREQ = 3
DENO = float(np.linalg.norm(np.array([0.004, 0.004, 0.004], dtype=np.float32)))

QB = 512
PB = 2048
BIG_IDX = np.int32(2**30)


def _topk_body(q_ref, p_ref, qsq_ref, psq_ref, out_ref, d2s_ref, rv_ref, ri_ref):
    j = pl.program_id(1)
    n_p = pl.num_programs(1)

    @pl.when(j == 0)
    def _init():
        rv_ref[...] = jnp.full((QB, K), jnp.inf, jnp.float32)
        ri_ref[...] = jnp.full((QB, K), BIG_IDX, jnp.int32)

    qq = q_ref[...]
    pp = p_ref[...]
    qp = lax.dot_general(qq, pp, (((1,), (1,)), ((), ())))
    d2 = (qsq_ref[...] + psq_ref[...][None, :]) - 2.0 * qp
    d2s_ref[...] = d2

    tau = rv_ref[...][:, K - 1:K]
    cnt = jnp.sum((d2 < tau).astype(jnp.int32), axis=1)
    needed = jnp.minimum(jnp.max(cnt), K)

    colbase = j * PB
    idx = lax.broadcasted_iota(jnp.int32, (QB, PB), 1) + colbase
    k8 = lax.broadcasted_iota(jnp.int32, (QB, K), 1)

    for t in range(K):
        @pl.when(t < needed)
        def _extract():
            d2c = d2s_ref[...]
            mv = jnp.min(d2c, axis=1, keepdims=True)
            mi = jnp.min(jnp.where(d2c == mv, idx, BIG_IDX), axis=1, keepdims=True)
            rv = rv_ref[...]
            ri = ri_ref[...]
            lexlt = (rv < mv) | ((rv == mv) & (ri < mi))
            pos = jnp.sum(lexlt.astype(jnp.int32), axis=1, keepdims=True)
            prev_v = jnp.concatenate([rv[:, :1], rv[:, :K - 1]], axis=1)
            prev_i = jnp.concatenate([ri[:, :1], ri[:, :K - 1]], axis=1)
            rv_ref[...] = jnp.where(k8 < pos, rv, jnp.where(k8 == pos, mv, prev_v))
            ri_ref[...] = jnp.where(k8 < pos, ri, jnp.where(k8 == pos, mi, prev_i))
            d2s_ref[...] = jnp.where(idx == mi, jnp.inf, d2c)

    @pl.when(j == n_p - 1)
    def _write():
        out_ref[...] = ri_ref[...]


def _knn_topk(query_points, neural_xyz, q_sq, p_sq):
    Q = query_points.shape[0]
    P = neural_xyz.shape[0]
    return pl.pallas_call(
        _topk_body,
        grid=(Q // QB, P // PB),
        in_specs=[
            pl.BlockSpec((QB, 3), lambda i, j: (i, 0)),
            pl.BlockSpec((PB, 3), lambda i, j: (j, 0)),
            pl.BlockSpec((QB, 1), lambda i, j: (i, 0)),
            pl.BlockSpec((PB,), lambda i, j: (j,)),
        ],
        out_specs=pl.BlockSpec((QB, K), lambda i, j: (i, 0)),
        out_shape=jax.ShapeDtypeStruct((Q, K), jnp.int32),
        scratch_shapes=[
            pltpu.VMEM((QB, PB), jnp.float32),
            pltpu.VMEM((QB, K), jnp.float32),
            pltpu.VMEM((QB, K), jnp.int32),
        ],
    )(query_points, neural_xyz, q_sq, p_sq)


_B_PER_W = 1024
_CHUNK = 128


def _gather_sc(table, idx_flat):
    B = idx_flat.shape[0]
    D = table.shape[1]
    mesh = plsc.VectorSubcoreMesh(core_axis_name="c", subcore_axis_name="s")

    @functools.partial(
        pl.kernel, mesh=mesh,
        out_type=jax.ShapeDtypeStruct((B, D), jnp.float32),
        scratch_types=[
            pltpu.VMEM((_CHUNK,), jnp.int32),
            pltpu.VMEM((_CHUNK, D), jnp.float32),
            pltpu.SemaphoreType.DMA,
        ],
    )
    def gk(table_hbm, idx_hbm, out_hbm, idx_v, rows_v, sem):
        wid = lax.axis_index("s") * 2 + lax.axis_index("c")
        for c in range(_B_PER_W // _CHUNK):
            base = wid * _B_PER_W + c * _CHUNK
            pltpu.sync_copy(idx_hbm.at[pl.ds(base, _CHUNK)], idx_v)
            pltpu.async_copy(table_hbm.at[idx_v], rows_v, sem).wait()
            pltpu.sync_copy(rows_v, out_hbm.at[pl.ds(base, _CHUNK)])

    return gk(table, idx_flat)


QB3 = 128

def _sel_mats():
    sd = np.zeros((6, 6 * DIST_FREQ), np.float32)
    for d in range(6):
        for f in range(DIST_FREQ):
            sd[d, d * DIST_FREQ + f] = 2.0 ** f
    sf = np.zeros((32, 32 * FEAT_FREQ), np.float32)
    for c in range(32):
        for f in range(FEAT_FREQ):
            sf[c, c * FEAT_FREQ + f] = 2.0 ** f
    return sd, sf

_SELD, _SELF = _sel_mats()


def _encode_body(g_ref, q_ref, cam_ref, campos_ref, rw2c_ref, seld_ref, self_ref, out_ref):
    g = g_ref[...]
    qq = q_ref[...]
    cam = cam_ref[...]
    campos = campos_ref[...]
    rw2c = rw2c_ref[...]
    N = QB3 * K

    def w2pers(xyz):
        sh = xyz - campos
        rot = (sh[:, 0:1] * cam[0:1, :] + sh[:, 1:2] * cam[1:2, :]
               + sh[:, 2:3] * cam[2:3, :])
        z = rot[:, 2:3]
        return rot[:, 0:1] / z, rot[:, 1:2] / z, z

    ref_xyz = g[:, 0:3]
    fea = g[:, 16:48]

    qx, qy, qz = w2pers(qq)
    rx, ry, rz = w2pers(ref_xyz)

    def rep(a):
        return jnp.broadcast_to(a[:, None, :], (QB3, K, 1)).reshape(N, 1)

    xdist = rx * rz - rep(qx * qz)
    ydist = ry * rz - rep(qy * qz)
    zdist = rz - rep(qz)

    qrep = jnp.broadcast_to(qq[:, None, :], (QB3, K, 3)).reshape(N, 3)
    dxyz = (ref_xyz - qrep) / DENO
    drot = lax.dot_general(dxyz, rw2c, (((1,), (0,)), ((), ())))
    dists = jnp.concatenate(
        [drot, xdist / DENO, ydist / DENO, zdist / DENO],
        axis=1)

    def pe_expand(x, sel):
        hi = x.astype(jnp.bfloat16).astype(jnp.float32)
        mid = (x - hi).astype(jnp.bfloat16).astype(jnp.float32)
        lo = x - hi - mid
        dims = (((1,), (0,)), ((), ()))
        return ((lax.dot_general(hi, sel, dims) + lax.dot_general(mid, sel, dims))
                + lax.dot_general(lo, sel, dims))

    dp = pe_expand(dists, seld_ref[...])
    fp = pe_expand(fea, self_ref[...])

    row = jnp.concatenate(
        [fea, jnp.sin(fp), jnp.cos(fp), jnp.sin(dp), jnp.cos(dp)], axis=1)
    out_ref[...] = row.reshape(QB3, K, 284)


def _encode(gathered, query_points, cam, campos, rw2c):
    Q = query_points.shape[0]
    return pl.pallas_call(
        _encode_body,
        grid=(Q // QB3,),
        in_specs=[
            pl.BlockSpec((QB3 * K, 128), lambda i: (i, 0)),
            pl.BlockSpec((QB3, 3), lambda i: (i, 0)),
            pl.BlockSpec((3, 3), lambda i: (0, 0)),
            pl.BlockSpec((1, 3), lambda i: (0, 0)),
            pl.BlockSpec((3, 3), lambda i: (0, 0)),
            pl.BlockSpec((6, 6 * DIST_FREQ), lambda i: (0, 0)),
            pl.BlockSpec((32, 32 * FEAT_FREQ), lambda i: (0, 0)),
        ],
        out_specs=pl.BlockSpec((QB3, K, 284), lambda i: (i, 0, 0)),
        out_shape=jax.ShapeDtypeStruct((Q, K, 284), jnp.float32),
    )(gathered, query_points, cam, campos, rw2c,
      jnp.asarray(_SELD), jnp.asarray(_SELF))


def kernel(query_points, neural_xyz, points_embedding, camrotc2w, campos, sampled_Rw2c):
    Q = query_points.shape[0]
    P = neural_xyz.shape[0]
    q_sq = jnp.sum(query_points * query_points, axis=-1, keepdims=True)
    p_sq = jnp.sum(neural_xyz * neural_xyz, axis=-1)

    assign_index = _knn_topk(query_points, neural_xyz, q_sq, p_sq)

    table = jnp.concatenate(
        [neural_xyz, jnp.zeros((P, 13), jnp.float32), points_embedding,
         jnp.zeros((P, 80), jnp.float32)], axis=1)
    gathered = _gather_sc(table, assign_index.reshape(-1))

    return _encode(gathered, query_points, camrotc2w.reshape(3, 3), campos,
                   sampled_Rw2c)


# speedup vs baseline: 3.2530x; 1.0734x over previous
import functools

import numpy as np
import jax
import jax.numpy as jnp
from jax import lax
from jax.experimental import pallas as pl
from jax.experimental.pallas import tpu as pltpu
from jax.experimental.pallas import tpu_sc as plsc

K = 8
DIST_FREQ = 5
FEAT_FREQ = 3
DENO = float(np.linalg.norm(np.array([0.004, 0.004, 0.004], dtype=np.float32)))

QB = 512
PB = 2048
BIG_IDX = np.int32(2**30)


def _topk_body(q_ref, p_ref, qsq_ref, psq_ref, out_ref,
               lov_ref, loi_ref, hiv_ref, hii_ref, rv_ref, ri_ref):
    j = pl.program_id(1)
    n_p = pl.num_programs(1)
    HB = PB // 2

    @pl.when(j == 0)
    def _init():
        rv_ref[...] = jnp.full((QB, K), jnp.inf, jnp.float32)
        ri_ref[...] = jnp.full((QB, K), BIG_IDX, jnp.int32)

    qq = q_ref[...]
    pp = p_ref[...]
    qp = lax.dot_general(qq, pp, (((1,), (1,)), ((), ())))
    d2 = (qsq_ref[...] + psq_ref[...][None, :]) - 2.0 * qp

    tau = rv_ref[...][:, K - 1:K]
    cnt = jnp.sum((d2 < tau).astype(jnp.int32), axis=1)
    needed = jnp.minimum(jnp.max(cnt), K)

    colbase = j * PB
    a = d2[:, :HB]
    b = d2[:, HB:]
    ia = lax.broadcasted_iota(jnp.int32, (QB, HB), 1) + colbase
    ib = ia + HB
    sel = a <= b
    lov_ref[...] = jnp.where(sel, a, b)
    loi_ref[...] = jnp.where(sel, ia, ib)
    hiv_ref[...] = jnp.where(sel, b, a)
    hii_ref[...] = jnp.where(sel, ib, ia)

    k8 = lax.broadcasted_iota(jnp.int32, (QB, K), 1)

    for t in range(K):
        @pl.when(t < needed)
        def _extract():
            lov = lov_ref[...]
            loi = loi_ref[...]
            mv = jnp.min(lov, axis=1, keepdims=True)
            mi = jnp.min(jnp.where(lov == mv, loi, BIG_IDX), axis=1, keepdims=True)
            rv = rv_ref[...]
            ri = ri_ref[...]
            lexlt = (rv < mv) | ((rv == mv) & (ri < mi))
            pos = jnp.sum(lexlt.astype(jnp.int32), axis=1, keepdims=True)
            prev_v = jnp.concatenate([rv[:, :1], rv[:, :K - 1]], axis=1)
            prev_i = jnp.concatenate([ri[:, :1], ri[:, :K - 1]], axis=1)
            rv_ref[...] = jnp.where(k8 < pos, rv, jnp.where(k8 == pos, mv, prev_v))
            ri_ref[...] = jnp.where(k8 < pos, ri, jnp.where(k8 == pos, mi, prev_i))
            hit = loi == mi
            lov_ref[...] = jnp.where(hit, hiv_ref[...], lov)
            loi_ref[...] = jnp.where(hit, hii_ref[...], loi)
            hiv_ref[...] = jnp.where(hit, jnp.inf, hiv_ref[...])

    @pl.when(j == n_p - 1)
    def _write():
        out_ref[...] = ri_ref[...]


def _knn_topk(query_points, neural_xyz, q_sq, p_sq):
    Q = query_points.shape[0]
    P = neural_xyz.shape[0]
    return pl.pallas_call(
        _topk_body,
        grid=(Q // QB, P // PB),
        in_specs=[
            pl.BlockSpec((QB, 3), lambda i, j: (i, 0)),
            pl.BlockSpec((PB, 3), lambda i, j: (j, 0)),
            pl.BlockSpec((QB, 1), lambda i, j: (i, 0)),
            pl.BlockSpec((PB,), lambda i, j: (j,)),
        ],
        out_specs=pl.BlockSpec((QB, K), lambda i, j: (i, 0)),
        out_shape=jax.ShapeDtypeStruct((Q, K), jnp.int32),
        scratch_shapes=[
            pltpu.VMEM((QB, PB // 2), jnp.float32),
            pltpu.VMEM((QB, PB // 2), jnp.int32),
            pltpu.VMEM((QB, PB // 2), jnp.float32),
            pltpu.VMEM((QB, PB // 2), jnp.int32),
            pltpu.VMEM((QB, K), jnp.float32),
            pltpu.VMEM((QB, K), jnp.int32),
        ],
    )(query_points, neural_xyz, q_sq, p_sq)


_B_PER_W = 1024
_CHUNK = 128


def _gather_sc(table, idx_flat):
    B = idx_flat.shape[0]
    D = table.shape[1]
    mesh = plsc.VectorSubcoreMesh(core_axis_name="c", subcore_axis_name="s")

    @functools.partial(
        pl.kernel, mesh=mesh,
        out_type=jax.ShapeDtypeStruct((B, D), jnp.float32),
        scratch_types=[
            pltpu.VMEM((_CHUNK,), jnp.int32),
            pltpu.VMEM((_CHUNK, D), jnp.float32),
            pltpu.SemaphoreType.DMA,
        ],
    )
    def gk(table_hbm, idx_hbm, out_hbm, idx_v, rows_v, sem):
        wid = lax.axis_index("s") * 2 + lax.axis_index("c")
        for c in range(_B_PER_W // _CHUNK):
            base = wid * _B_PER_W + c * _CHUNK
            pltpu.sync_copy(idx_hbm.at[pl.ds(base, _CHUNK)], idx_v)
            pltpu.async_copy(table_hbm.at[idx_v], rows_v, sem).wait()
            pltpu.sync_copy(rows_v, out_hbm.at[pl.ds(base, _CHUNK)])

    return gk(table, idx_flat)


QB3 = 128

def _sel_mats():
    sd = np.zeros((6, 6 * DIST_FREQ), np.float32)
    for d in range(6):
        for f in range(DIST_FREQ):
            sd[d, d * DIST_FREQ + f] = 2.0 ** f
    sf = np.zeros((32, 32 * FEAT_FREQ), np.float32)
    for c in range(32):
        for f in range(FEAT_FREQ):
            sf[c, c * FEAT_FREQ + f] = 2.0 ** f
    return sd, sf

_SELD, _SELF = _sel_mats()


def _encode_body(g_ref, q_ref, cam_ref, campos_ref, rw2c_ref, seld_ref, self_ref, out_ref):
    g = g_ref[...]
    qq = q_ref[...]
    cam = cam_ref[...]
    campos = campos_ref[...]
    rw2c = rw2c_ref[...]
    N = QB3 * K

    def w2pers(xyz):
        sh = xyz - campos
        rot = (sh[:, 0:1] * cam[0:1, :] + sh[:, 1:2] * cam[1:2, :]
               + sh[:, 2:3] * cam[2:3, :])
        z = rot[:, 2:3]
        return rot[:, 0:1] / z, rot[:, 1:2] / z, z

    ref_xyz = g[:, 0:3]
    fea = g[:, 16:48]

    qx, qy, qz = w2pers(qq)
    rx, ry, rz = w2pers(ref_xyz)

    def rep(a):
        return jnp.broadcast_to(a[:, None, :], (QB3, K, 1)).reshape(N, 1)

    xdist = rx * rz - rep(qx * qz)
    ydist = ry * rz - rep(qy * qz)
    zdist = rz - rep(qz)

    qrep = jnp.broadcast_to(qq[:, None, :], (QB3, K, 3)).reshape(N, 3)
    dxyz = (ref_xyz - qrep) / DENO
    drot = lax.dot_general(dxyz, rw2c, (((1,), (0,)), ((), ())))
    dists = jnp.concatenate(
        [drot, xdist / DENO, ydist / DENO, zdist / DENO],
        axis=1)

    def pe_expand(x, sel):
        hi = x.astype(jnp.bfloat16).astype(jnp.float32)
        mid = (x - hi).astype(jnp.bfloat16).astype(jnp.float32)
        lo = x - hi - mid
        dims = (((1,), (0,)), ((), ()))
        return ((lax.dot_general(hi, sel, dims) + lax.dot_general(mid, sel, dims))
                + lax.dot_general(lo, sel, dims))

    dp = pe_expand(dists, seld_ref[...])
    fp = pe_expand(fea, self_ref[...])

    row = jnp.concatenate(
        [fea, jnp.sin(fp), jnp.cos(fp), jnp.sin(dp), jnp.cos(dp)], axis=1)
    out_ref[...] = row.reshape(QB3, K, 284)


def _encode(gathered, query_points, cam, campos, rw2c):
    Q = query_points.shape[0]
    return pl.pallas_call(
        _encode_body,
        grid=(Q // QB3,),
        in_specs=[
            pl.BlockSpec((QB3 * K, 128), lambda i: (i, 0)),
            pl.BlockSpec((QB3, 3), lambda i: (i, 0)),
            pl.BlockSpec((3, 3), lambda i: (0, 0)),
            pl.BlockSpec((1, 3), lambda i: (0, 0)),
            pl.BlockSpec((3, 3), lambda i: (0, 0)),
            pl.BlockSpec((6, 6 * DIST_FREQ), lambda i: (0, 0)),
            pl.BlockSpec((32, 32 * FEAT_FREQ), lambda i: (0, 0)),
        ],
        out_specs=pl.BlockSpec((QB3, K, 284), lambda i: (i, 0, 0)),
        out_shape=jax.ShapeDtypeStruct((Q, K, 284), jnp.float32),
    )(gathered, query_points, cam, campos, rw2c,
      jnp.asarray(_SELD), jnp.asarray(_SELF))


def kernel(query_points, neural_xyz, points_embedding, camrotc2w, campos, sampled_Rw2c):
    Q = query_points.shape[0]
    P = neural_xyz.shape[0]
    q_sq = jnp.sum(query_points * query_points, axis=-1, keepdims=True)
    p_sq = jnp.sum(neural_xyz * neural_xyz, axis=-1)

    assign_index = _knn_topk(query_points, neural_xyz, q_sq, p_sq)

    table = jnp.concatenate(
        [neural_xyz, jnp.zeros((P, 13), jnp.float32), points_embedding,
         jnp.zeros((P, 80), jnp.float32)], axis=1)
    gathered = _gather_sc(table, assign_index.reshape(-1))

    return _encode(gathered, query_points, camrotc2w.reshape(3, 3), campos,
                   sampled_Rw2c)
